# baseline (speedup 1.0000x reference)
import functools

import jax
import jax.numpy as jnp
from jax import lax
from jax.experimental import pallas as pl
from jax.experimental.pallas import tpu as pltpu
from jax.experimental.pallas import tpu_sc as plsc

N = 10000
D = 128
E = 320000
NC = 2
NS = 16
NW = NC * NS
EPW = E // NW
K = 80
CHUNKS = EPW // K
NP = 10240
RPT = NP // NS
ZR = 16


def _sc_agg_body(h_hbm, src_hbm, dst_hbm, acc_out, deg_out,
                 idxa_v, idxb_v, idxc_v, rows_v, ones_v, zbuf_v, zdeg_v,
                 acc_sh, deg_sh, sem0, sem1, sem2):
    idxbufs = (idxa_v, idxb_v, idxc_v)
    gsems = (sem0, sem1, sem2)
    c = lax.axis_index("c")
    s = lax.axis_index("s")
    wid = c * NS + s

    zv = jnp.zeros((16,), jnp.float32)
    ov = jnp.ones((16,), jnp.float32)

    @pl.loop(0, ZR)
    def _zero_bufs(i):
        for j in range(D // 16):
            zbuf_v[i, pl.ds(j * 16, 16)] = zv

    @pl.loop(0, RPT // 16)
    def _zero_deg(i):
        zdeg_v[pl.ds(i * 16, 16)] = zv

    @pl.loop(0, K // 16)
    def _init_ones(i):
        ones_v[pl.ds(i * 16, 16)] = ov

    row0 = s * RPT
    for t in range(RPT // ZR):
        pltpu.sync_copy(zbuf_v, acc_sh.at[pl.ds(row0 + t * ZR, ZR)])
    pltpu.sync_copy(zdeg_v, deg_sh.at[pl.ds(row0, RPT)])
    plsc.subcore_barrier()

    base = wid * EPW

    def load_idx(j, buf):
        off = base + j * K
        pltpu.sync_copy(src_hbm.at[pl.ds(off, K)], buf.at[0])
        pltpu.sync_copy(dst_hbm.at[pl.ds(off, K)], buf.at[1])

    def fire_gather(buf, slot):
        pltpu.async_copy(h_hbm.at[buf.at[0]], rows_v.at[slot], gsems[slot])

    def consume(buf, slot):
        pltpu.make_async_copy(h_hbm.at[buf.at[0]], rows_v.at[slot],
                              gsems[slot]).wait()
        pltpu.sync_copy(rows_v.at[slot], acc_sh.at[buf.at[1]], add=True)
        pltpu.sync_copy(ones_v, deg_sh.at[buf.at[1]], add=True)

    for j in range(2):
        load_idx(j, idxbufs[j])
        fire_gather(idxbufs[j], j)

    @pl.loop(0, (CHUNKS - 2) // 3)
    def _edges(i):
        for t in range(3):
            j = i * 3 + t + 2
            sj = (t + 2) % 3
            load_idx(j, idxbufs[sj])
            fire_gather(idxbufs[sj], sj)
            consume(idxbufs[(sj + 1) % 3], (sj + 1) % 3)

    consume(idxbufs[(CHUNKS - 2) % 3], (CHUNKS - 2) % 3)
    consume(idxbufs[(CHUNKS - 1) % 3], (CHUNKS - 1) % 3)

    plsc.subcore_barrier()
    pltpu.sync_copy(acc_sh.at[pl.ds(row0, RPT)], acc_out.at[c, pl.ds(row0, RPT)])
    pltpu.sync_copy(deg_sh.at[pl.ds(row0, RPT)], deg_out.at[c, pl.ds(row0, RPT)])


@functools.lru_cache(maxsize=None)
def _make_sc_agg():
    return pl.kernel(
        _sc_agg_body,
        out_type=(
            jax.ShapeDtypeStruct((NC, NP, D), jnp.float32),
            jax.ShapeDtypeStruct((NC, NP), jnp.float32),
        ),
        mesh=plsc.VectorSubcoreMesh(core_axis_name="c", subcore_axis_name="s",
                                    num_cores=NC, num_subcores=NS),
        scratch_types=[
            pltpu.VMEM((2, K), jnp.int32),
            pltpu.VMEM((2, K), jnp.int32),
            pltpu.VMEM((2, K), jnp.int32),
            pltpu.VMEM((3, K, D), jnp.float32),
            pltpu.VMEM((K,), jnp.float32),
            pltpu.VMEM((ZR, D), jnp.float32),
            pltpu.VMEM((RPT,), jnp.float32),
            pltpu.VMEM_SHARED((NP, D), jnp.float32),
            pltpu.VMEM_SHARED((NP,), jnp.float32),
            pltpu.SemaphoreType.DMA,
            pltpu.SemaphoreType.DMA,
            pltpu.SemaphoreType.DMA,
        ],
    )


def _mm_body(relu, x_ref, a0_ref, a1_ref, d0_ref, d1_ref,
             ws_ref, wn_ref, b_ref, o_ref):
    x = x_ref[...]
    a = a0_ref[...] + a1_ref[...]
    deg = jnp.clip(d0_ref[...] + d1_ref[...], 1.0, None)
    mean = a / deg
    out = (jnp.dot(x, ws_ref[...], preferred_element_type=jnp.float32)
           + jnp.dot(mean, wn_ref[...], preferred_element_type=jnp.float32)
           + b_ref[...])
    if relu:
        out = jnp.maximum(out, 0.0)
    o_ref[...] = out


def _mm(relu, x, a0, a1, d0, d1, ws, wn, b):
    R = 1000
    grid = (N // R,)
    return pl.pallas_call(
        functools.partial(_mm_body, relu),
        grid=grid,
        in_specs=[
            pl.BlockSpec((R, D), lambda i: (i, 0)),
            pl.BlockSpec((R, D), lambda i: (i, 0)),
            pl.BlockSpec((R, D), lambda i: (i, 0)),
            pl.BlockSpec((R, 1), lambda i: (i, 0)),
            pl.BlockSpec((R, 1), lambda i: (i, 0)),
            pl.BlockSpec((D, D), lambda i: (0, 0)),
            pl.BlockSpec((D, D), lambda i: (0, 0)),
            pl.BlockSpec((1, D), lambda i: (0, 0)),
        ],
        out_specs=pl.BlockSpec((R, D), lambda i: (i, 0)),
        out_shape=jax.ShapeDtypeStruct((N, D), jnp.float32),
    )(x, a0, a1, d0, d1, ws, wn, b)


def kernel(input_features, edge_index0, edge_index1,
           W_self0, W_neigh0, b0, W_self1, W_neigh1, b1):
    src0 = edge_index0[0].astype(jnp.int32)
    dst0 = edge_index0[1].astype(jnp.int32)
    src1 = edge_index1[0].astype(jnp.int32)
    dst1 = edge_index1[1].astype(jnp.int32)

    sc_agg = _make_sc_agg()
    acc0, deg0 = sc_agg(input_features, src0, dst0)
    h1 = _mm(True, input_features, acc0[0, :N], acc0[1, :N],
             deg0[0, :N].reshape(N, 1), deg0[1, :N].reshape(N, 1),
             W_self0, W_neigh0, b0.reshape(1, D))
    acc1, deg1 = sc_agg(h1, src1, dst1)
    return _mm(False, h1, acc1[0, :N], acc1[1, :N],
               deg1[0, :N].reshape(N, 1), deg1[1, :N].reshape(N, 1),
               W_self1, W_neigh1, b1.reshape(1, D))


# speedup vs baseline: 1.2239x; 1.2239x over previous
import functools

import jax
import jax.numpy as jnp
from jax import lax
from jax.experimental import pallas as pl
from jax.experimental.pallas import tpu as pltpu
from jax.experimental.pallas import tpu_sc as plsc

N = 10000
D = 128
E = 320000
NC = 2
NS = 16
NW = NC * NS
EPW = E // NW
K = 80
CHUNKS = EPW // K
NP = 10240
RPT = NP // NS
ZR = 16


def _sc_agg_body(h_hbm, src_hbm, dst_hbm, acc_out, deg_out,
                 idxa_v, idxb_v, idxc_v, rows_v, ones_v, zbuf_v, zdeg_v,
                 acc_sh, deg_sh, *sems):
    idxbufs = (idxa_v, idxb_v, idxc_v)
    gsems = sems[0:3]
    asems = sems[3:6]
    dsems = sems[6:9]
    c = lax.axis_index("c")
    s = lax.axis_index("s")
    wid = c * NS + s

    zv = jnp.zeros((16,), jnp.float32)
    ov = jnp.ones((16,), jnp.float32)

    @pl.loop(0, ZR)
    def _zero_bufs(i):
        for j in range(D // 16):
            zbuf_v[i, pl.ds(j * 16, 16)] = zv

    @pl.loop(0, RPT // 16)
    def _zero_deg(i):
        zdeg_v[pl.ds(i * 16, 16)] = zv

    @pl.loop(0, K // 16)
    def _init_ones(i):
        ones_v[pl.ds(i * 16, 16)] = ov

    row0 = s * RPT
    for t in range(RPT // ZR):
        pltpu.sync_copy(zbuf_v, acc_sh.at[pl.ds(row0 + t * ZR, ZR)])
    pltpu.sync_copy(zdeg_v, deg_sh.at[pl.ds(row0, RPT)])
    plsc.subcore_barrier()

    base = wid * EPW

    def load_idx(j, slot):
        off = base + j * K
        pltpu.sync_copy(src_hbm.at[pl.ds(off, K)], idxbufs[slot].at[0])
        pltpu.sync_copy(dst_hbm.at[pl.ds(off, K)], idxbufs[slot].at[1])

    def fire_gather(slot):
        pltpu.async_copy(h_hbm.at[idxbufs[slot].at[0]], rows_v.at[slot],
                         gsems[slot])

    def wait_gather(slot):
        pltpu.make_async_copy(h_hbm.at[idxbufs[slot].at[0]],
                              rows_v.at[slot], gsems[slot]).wait()

    def fire_scatter(slot):
        pltpu.async_copy(rows_v.at[slot], acc_sh.at[idxbufs[slot].at[1]],
                         asems[slot], add=True)
        pltpu.async_copy(ones_v, deg_sh.at[idxbufs[slot].at[1]],
                         dsems[slot], add=True)

    def wait_scatter(slot):
        pltpu.make_async_copy(rows_v.at[slot],
                              acc_sh.at[idxbufs[slot].at[1]],
                              asems[slot]).wait()
        pltpu.make_async_copy(ones_v, deg_sh.at[idxbufs[slot].at[1]],
                              dsems[slot]).wait()

    load_idx(0, 0)
    fire_gather(0)

    @pl.loop(0, (CHUNKS - 2) // 3)
    def _edges(i):
        for t in range(3):
            j = i * 3 + t + 1
            sj = (t + 1) % 3

            @pl.when(j >= 3)
            def _drain():
                wait_scatter(sj)

            load_idx(j, sj)
            fire_gather(sj)
            wait_gather((sj + 2) % 3)
            fire_scatter((sj + 2) % 3)

    j = CHUNKS - 1
    sj = j % 3
    wait_scatter(sj)
    load_idx(j, sj)
    fire_gather(sj)
    wait_gather((sj + 2) % 3)
    fire_scatter((sj + 2) % 3)
    wait_gather(sj)
    fire_scatter(sj)
    wait_scatter((sj + 1) % 3)
    wait_scatter((sj + 2) % 3)
    wait_scatter(sj)

    plsc.subcore_barrier()
    pltpu.sync_copy(acc_sh.at[pl.ds(row0, RPT)], acc_out.at[c, pl.ds(row0, RPT)])
    pltpu.sync_copy(deg_sh.at[pl.ds(row0, RPT)], deg_out.at[c, pl.ds(row0, RPT)])


@functools.lru_cache(maxsize=None)
def _make_sc_agg():
    return pl.kernel(
        _sc_agg_body,
        out_type=(
            jax.ShapeDtypeStruct((NC, NP, D), jnp.float32),
            jax.ShapeDtypeStruct((NC, NP), jnp.float32),
        ),
        mesh=plsc.VectorSubcoreMesh(core_axis_name="c", subcore_axis_name="s",
                                    num_cores=NC, num_subcores=NS),
        scratch_types=[
            pltpu.VMEM((2, K), jnp.int32),
            pltpu.VMEM((2, K), jnp.int32),
            pltpu.VMEM((2, K), jnp.int32),
            pltpu.VMEM((3, K, D), jnp.float32),
            pltpu.VMEM((K,), jnp.float32),
            pltpu.VMEM((ZR, D), jnp.float32),
            pltpu.VMEM((RPT,), jnp.float32),
            pltpu.VMEM_SHARED((NP, D), jnp.float32),
            pltpu.VMEM_SHARED((NP,), jnp.float32),
        ] + [pltpu.SemaphoreType.DMA] * 9,
    )


def _mm_body(relu, x_ref, a0_ref, a1_ref, d0_ref, d1_ref,
             ws_ref, wn_ref, b_ref, o_ref):
    x = x_ref[...]
    a = a0_ref[...] + a1_ref[...]
    deg = jnp.clip(d0_ref[...] + d1_ref[...], 1.0, None)
    mean = a / deg
    out = (jnp.dot(x, ws_ref[...], preferred_element_type=jnp.float32)
           + jnp.dot(mean, wn_ref[...], preferred_element_type=jnp.float32)
           + b_ref[...])
    if relu:
        out = jnp.maximum(out, 0.0)
    o_ref[...] = out


def _mm(relu, x, a0, a1, d0, d1, ws, wn, b):
    R = 1000
    grid = (N // R,)
    return pl.pallas_call(
        functools.partial(_mm_body, relu),
        grid=grid,
        in_specs=[
            pl.BlockSpec((R, D), lambda i: (i, 0)),
            pl.BlockSpec((R, D), lambda i: (i, 0)),
            pl.BlockSpec((R, D), lambda i: (i, 0)),
            pl.BlockSpec((R, 1), lambda i: (i, 0)),
            pl.BlockSpec((R, 1), lambda i: (i, 0)),
            pl.BlockSpec((D, D), lambda i: (0, 0)),
            pl.BlockSpec((D, D), lambda i: (0, 0)),
            pl.BlockSpec((1, D), lambda i: (0, 0)),
        ],
        out_specs=pl.BlockSpec((R, D), lambda i: (i, 0)),
        out_shape=jax.ShapeDtypeStruct((N, D), jnp.float32),
    )(x, a0, a1, d0, d1, ws, wn, b)


def kernel(input_features, edge_index0, edge_index1,
           W_self0, W_neigh0, b0, W_self1, W_neigh1, b1):
    src0 = edge_index0[0].astype(jnp.int32)
    dst0 = edge_index0[1].astype(jnp.int32)
    src1 = edge_index1[0].astype(jnp.int32)
    dst1 = edge_index1[1].astype(jnp.int32)

    sc_agg = _make_sc_agg()
    acc0, deg0 = sc_agg(input_features, src0, dst0)
    h1 = _mm(True, input_features, acc0[0, :N], acc0[1, :N],
             deg0[0, :N].reshape(N, 1), deg0[1, :N].reshape(N, 1),
             W_self0, W_neigh0, b0.reshape(1, D))
    acc1, deg1 = sc_agg(h1, src1, dst1)
    return _mm(False, h1, acc1[0, :N], acc1[1, :N],
               deg1[0, :N].reshape(N, 1), deg1[1, :N].reshape(N, 1),
               W_self1, W_neigh1, b1.reshape(1, D))


# speedup vs baseline: 1.6300x; 1.3318x over previous
import functools

import jax
import jax.numpy as jnp
from jax import lax
from jax.experimental import pallas as pl
from jax.experimental.pallas import tpu as pltpu
from jax.experimental.pallas import tpu_sc as plsc

N = 10000
D = 128
E = 320000
NC = 2
NS = 16
NW = NC * NS
EPW = E // NW
K = 80
CHUNKS = EPW // K
NP = 10240
RPT = NP // NS
ZR = 16


def _sc_agg_body(h_hbm, src_hbm, dst_hbm, acc_out, deg_out,
                 ix0, ix1, ix2, ix3, ix4, ix5, rows_v, ones_v, zbuf_v,
                 zdeg_v, acc_sh, deg_sh, *sems):
    idxbufs = (ix0, ix1, ix2, ix3, ix4, ix5)
    gsems = sems[0:3]
    asems = sems[3:6]
    dsems = sems[6:9]
    isems = sems[9:15]
    c = lax.axis_index("c")
    s = lax.axis_index("s")
    wid = c * NS + s

    zv = jnp.zeros((16,), jnp.float32)
    ov = jnp.ones((16,), jnp.float32)

    @pl.loop(0, ZR)
    def _zero_bufs(i):
        for j in range(D // 16):
            zbuf_v[i, pl.ds(j * 16, 16)] = zv

    @pl.loop(0, RPT // 16)
    def _zero_deg(i):
        zdeg_v[pl.ds(i * 16, 16)] = zv

    @pl.loop(0, K // 16)
    def _init_ones(i):
        ones_v[pl.ds(i * 16, 16)] = ov

    row0 = s * RPT
    for t in range(RPT // ZR):
        pltpu.sync_copy(zbuf_v, acc_sh.at[pl.ds(row0 + t * ZR, ZR)])
    pltpu.sync_copy(zdeg_v, deg_sh.at[pl.ds(row0, RPT)])
    plsc.subcore_barrier()

    base = wid * EPW

    def fire_idx(j, islot):
        off = base + j * K
        pltpu.async_copy(src_hbm.at[pl.ds(off, K)], idxbufs[islot].at[0],
                         isems[islot])
        pltpu.async_copy(dst_hbm.at[pl.ds(off, K)], idxbufs[islot].at[1],
                         isems[islot])

    def wait_idx(j, islot):
        off = base + j * K
        pltpu.make_async_copy(src_hbm.at[pl.ds(off, K)],
                              idxbufs[islot].at[0], isems[islot]).wait()
        pltpu.make_async_copy(dst_hbm.at[pl.ds(off, K)],
                              idxbufs[islot].at[1], isems[islot]).wait()

    def fire_gather(slot, islot):
        pltpu.async_copy(h_hbm.at[idxbufs[islot].at[0]], rows_v.at[slot],
                         gsems[slot])

    def wait_gather(slot, islot):
        pltpu.make_async_copy(h_hbm.at[idxbufs[islot].at[0]],
                              rows_v.at[slot], gsems[slot]).wait()

    def fire_scatter(slot, islot):
        pltpu.async_copy(rows_v.at[slot], acc_sh.at[idxbufs[islot].at[1]],
                         asems[slot], add=True)
        pltpu.async_copy(ones_v, deg_sh.at[idxbufs[islot].at[1]],
                         dsems[slot], add=True)

    def wait_scatter(slot, islot):
        pltpu.make_async_copy(rows_v.at[slot],
                              acc_sh.at[idxbufs[islot].at[1]],
                              asems[slot]).wait()
        pltpu.make_async_copy(ones_v, deg_sh.at[idxbufs[islot].at[1]],
                              dsems[slot]).wait()

    for m in range(4):
        fire_idx(m, m)
    wait_idx(0, 0)
    fire_gather(0, 0)

    @pl.loop(0, (CHUNKS - 5) // 6)
    def _edges(i):
        for t in range(6):
            j = i * 6 + t + 1
            sj = (t + 1) % 3
            ij = (t + 1) % 6

            @pl.when(j >= 3)
            def _drain():
                wait_scatter(sj, (ij + 3) % 6)

            fire_idx(j + 3, (ij + 3) % 6)
            wait_idx(j, ij)
            fire_gather(sj, ij)
            wait_gather((sj + 2) % 3, (ij + 5) % 6)
            fire_scatter((sj + 2) % 3, (ij + 5) % 6)

    for j in range(CHUNKS - 4, CHUNKS):
        sj = j % 3
        ij = j % 6
        wait_scatter(sj, (ij + 3) % 6)
        if j + 3 < CHUNKS:
            fire_idx(j + 3, (ij + 3) % 6)
        wait_idx(j, ij)
        fire_gather(sj, ij)
        wait_gather((sj + 2) % 3, (ij + 5) % 6)
        fire_scatter((sj + 2) % 3, (ij + 5) % 6)

    j = CHUNKS - 1
    wait_gather(j % 3, j % 6)
    fire_scatter(j % 3, j % 6)
    for m in range(CHUNKS - 3, CHUNKS):
        wait_scatter(m % 3, m % 6)

    plsc.subcore_barrier()
    pltpu.sync_copy(acc_sh.at[pl.ds(row0, RPT)], acc_out.at[c, pl.ds(row0, RPT)])
    pltpu.sync_copy(deg_sh.at[pl.ds(row0, RPT)], deg_out.at[c, pl.ds(row0, RPT)])


@functools.lru_cache(maxsize=None)
def _make_sc_agg():
    return pl.kernel(
        _sc_agg_body,
        out_type=(
            jax.ShapeDtypeStruct((NC, NP, D), jnp.float32),
            jax.ShapeDtypeStruct((NC, NP), jnp.float32),
        ),
        mesh=plsc.VectorSubcoreMesh(core_axis_name="c", subcore_axis_name="s",
                                    num_cores=NC, num_subcores=NS),
        scratch_types=[pltpu.VMEM((2, K), jnp.int32)] * 6 + [
            pltpu.VMEM((3, K, D), jnp.float32),
            pltpu.VMEM((K,), jnp.float32),
            pltpu.VMEM((ZR, D), jnp.float32),
            pltpu.VMEM((RPT,), jnp.float32),
            pltpu.VMEM_SHARED((NP, D), jnp.float32),
            pltpu.VMEM_SHARED((NP,), jnp.float32),
        ] + [pltpu.SemaphoreType.DMA] * 15,
    )


def _mm_body(relu, x_ref, a0_ref, a1_ref, d0_ref, d1_ref,
             ws_ref, wn_ref, b_ref, o_ref):
    x = x_ref[...]
    a = a0_ref[...] + a1_ref[...]
    deg = jnp.clip(d0_ref[...] + d1_ref[...], 1.0, None)
    mean = a / deg
    out = (jnp.dot(x, ws_ref[...], preferred_element_type=jnp.float32)
           + jnp.dot(mean, wn_ref[...], preferred_element_type=jnp.float32)
           + b_ref[...])
    if relu:
        out = jnp.maximum(out, 0.0)
    o_ref[...] = out


def _mm(relu, x, a0, a1, d0, d1, ws, wn, b):
    R = 1000
    grid = (N // R,)
    return pl.pallas_call(
        functools.partial(_mm_body, relu),
        grid=grid,
        in_specs=[
            pl.BlockSpec((R, D), lambda i: (i, 0)),
            pl.BlockSpec((R, D), lambda i: (i, 0)),
            pl.BlockSpec((R, D), lambda i: (i, 0)),
            pl.BlockSpec((R, 1), lambda i: (i, 0)),
            pl.BlockSpec((R, 1), lambda i: (i, 0)),
            pl.BlockSpec((D, D), lambda i: (0, 0)),
            pl.BlockSpec((D, D), lambda i: (0, 0)),
            pl.BlockSpec((1, D), lambda i: (0, 0)),
        ],
        out_specs=pl.BlockSpec((R, D), lambda i: (i, 0)),
        out_shape=jax.ShapeDtypeStruct((N, D), jnp.float32),
    )(x, a0, a1, d0, d1, ws, wn, b)


def kernel(input_features, edge_index0, edge_index1,
           W_self0, W_neigh0, b0, W_self1, W_neigh1, b1):
    src0 = edge_index0[0].astype(jnp.int32)
    dst0 = edge_index0[1].astype(jnp.int32)
    src1 = edge_index1[0].astype(jnp.int32)
    dst1 = edge_index1[1].astype(jnp.int32)

    sc_agg = _make_sc_agg()
    acc0, deg0 = sc_agg(input_features, src0, dst0)
    h1 = _mm(True, input_features, acc0[0, :N], acc0[1, :N],
             deg0[0, :N].reshape(N, 1), deg0[1, :N].reshape(N, 1),
             W_self0, W_neigh0, b0.reshape(1, D))
    acc1, deg1 = sc_agg(h1, src1, dst1)
    return _mm(False, h1, acc1[0, :N], acc1[1, :N],
               deg1[0, :N].reshape(N, 1), deg1[1, :N].reshape(N, 1),
               W_self1, W_neigh1, b1.reshape(1, D))


# speedup vs baseline: 1.6535x; 1.0145x over previous
import functools

import jax
import jax.numpy as jnp
from jax import lax
from jax.experimental import pallas as pl
from jax.experimental.pallas import tpu as pltpu
from jax.experimental.pallas import tpu_sc as plsc

N = 10000
D = 128
E = 320000
NC = 2
NS = 16
NW = NC * NS
EPW = E // NW
K = 80
CHUNKS = EPW // K
NP = 10240
RPT = NP // NS
ZR = 16


def _sc_agg_body(h_hbm, src_hbm, dst_hbm, acc_out, deg_out,
                 ix0, ix1, ix2, ix3, ix4, ix5, rows_v, ones_v, zbuf_v,
                 zdeg_v, acc_sh, deg_sh, *sems):
    idxbufs = (ix0, ix1, ix2, ix3, ix4, ix5)
    gsems = sems[0:3]
    asems = sems[3:6]
    dsems = sems[6:9]
    isems = sems[9:15]
    c = lax.axis_index("c")
    s = lax.axis_index("s")
    wid = c * NS + s

    zv = jnp.zeros((16,), jnp.float32)
    ov = jnp.ones((16,), jnp.float32)

    @pl.loop(0, ZR)
    def _zero_bufs(i):
        for j in range(D // 16):
            zbuf_v[i, pl.ds(j * 16, 16)] = zv

    @pl.loop(0, RPT // 16)
    def _zero_deg(i):
        zdeg_v[pl.ds(i * 16, 16)] = zv

    @pl.loop(0, K // 16)
    def _init_ones(i):
        ones_v[pl.ds(i * 16, 16)] = ov

    row0 = s * RPT
    for t in range(RPT // ZR):
        pltpu.async_copy(zbuf_v, acc_sh.at[pl.ds(row0 + t * ZR, ZR)],
                         sems[0])
    pltpu.sync_copy(zdeg_v, deg_sh.at[pl.ds(row0, RPT)])
    for t in range(RPT // ZR):
        pltpu.make_async_copy(zbuf_v, acc_sh.at[pl.ds(row0 + t * ZR, ZR)],
                              sems[0]).wait()
    plsc.subcore_barrier()

    base = wid * EPW

    def fire_idx(j, islot):
        off = base + j * K
        pltpu.async_copy(src_hbm.at[pl.ds(off, K)], idxbufs[islot].at[0],
                         isems[islot])
        pltpu.async_copy(dst_hbm.at[pl.ds(off, K)], idxbufs[islot].at[1],
                         isems[islot])

    def wait_idx(j, islot):
        off = base + j * K
        pltpu.make_async_copy(src_hbm.at[pl.ds(off, K)],
                              idxbufs[islot].at[0], isems[islot]).wait()
        pltpu.make_async_copy(dst_hbm.at[pl.ds(off, K)],
                              idxbufs[islot].at[1], isems[islot]).wait()

    def fire_gather(slot, islot):
        pltpu.async_copy(h_hbm.at[idxbufs[islot].at[0]], rows_v.at[slot],
                         gsems[slot])

    def wait_gather(slot, islot):
        pltpu.make_async_copy(h_hbm.at[idxbufs[islot].at[0]],
                              rows_v.at[slot], gsems[slot]).wait()

    def fire_scatter(slot, islot):
        pltpu.async_copy(rows_v.at[slot], acc_sh.at[idxbufs[islot].at[1]],
                         asems[slot], add=True)
        pltpu.async_copy(ones_v, deg_sh.at[idxbufs[islot].at[1]],
                         dsems[slot], add=True)

    def wait_scatter(slot, islot):
        pltpu.make_async_copy(rows_v.at[slot],
                              acc_sh.at[idxbufs[islot].at[1]],
                              asems[slot]).wait()
        pltpu.make_async_copy(ones_v, deg_sh.at[idxbufs[islot].at[1]],
                              dsems[slot]).wait()

    for m in range(4):
        fire_idx(m, m)
    wait_idx(0, 0)
    fire_gather(0, 0)

    @pl.loop(0, (CHUNKS - 5) // 6)
    def _edges(i):
        for t in range(6):
            j = i * 6 + t + 1
            sj = (t + 1) % 3
            ij = (t + 1) % 6

            @pl.when(j >= 3)
            def _drain():
                wait_scatter(sj, (ij + 3) % 6)

            fire_idx(j + 3, (ij + 3) % 6)
            wait_idx(j, ij)
            fire_gather(sj, ij)
            wait_gather((sj + 2) % 3, (ij + 5) % 6)
            fire_scatter((sj + 2) % 3, (ij + 5) % 6)

    for j in range(CHUNKS - 4, CHUNKS):
        sj = j % 3
        ij = j % 6
        wait_scatter(sj, (ij + 3) % 6)
        if j + 3 < CHUNKS:
            fire_idx(j + 3, (ij + 3) % 6)
        wait_idx(j, ij)
        fire_gather(sj, ij)
        wait_gather((sj + 2) % 3, (ij + 5) % 6)
        fire_scatter((sj + 2) % 3, (ij + 5) % 6)

    j = CHUNKS - 1
    wait_gather(j % 3, j % 6)
    fire_scatter(j % 3, j % 6)
    for m in range(CHUNKS - 3, CHUNKS):
        wait_scatter(m % 3, m % 6)

    plsc.subcore_barrier()
    pltpu.sync_copy(acc_sh.at[pl.ds(row0, RPT)], acc_out.at[c, pl.ds(row0, RPT)])
    pltpu.sync_copy(deg_sh.at[pl.ds(row0, RPT)], deg_out.at[c, pl.ds(row0, RPT)])


@functools.lru_cache(maxsize=None)
def _make_sc_agg():
    return pl.kernel(
        _sc_agg_body,
        out_type=(
            jax.ShapeDtypeStruct((NC, NP, D), jnp.float32),
            jax.ShapeDtypeStruct((NC, NP), jnp.float32),
        ),
        mesh=plsc.VectorSubcoreMesh(core_axis_name="c", subcore_axis_name="s",
                                    num_cores=NC, num_subcores=NS),
        scratch_types=[pltpu.VMEM((2, K), jnp.int32)] * 6 + [
            pltpu.VMEM((3, K, D), jnp.float32),
            pltpu.VMEM((K,), jnp.float32),
            pltpu.VMEM((ZR, D), jnp.float32),
            pltpu.VMEM((RPT,), jnp.float32),
            pltpu.VMEM_SHARED((NP, D), jnp.float32),
            pltpu.VMEM_SHARED((NP,), jnp.float32),
        ] + [pltpu.SemaphoreType.DMA] * 15,
    )


def _mm_body(relu, x_ref, a0_ref, a1_ref, d0_ref, d1_ref,
             ws_ref, wn_ref, b_ref, o_ref):
    x = x_ref[...]
    a = a0_ref[...] + a1_ref[...]
    deg = jnp.clip(d0_ref[...] + d1_ref[...], 1.0, None)
    mean = a / deg
    out = (jnp.dot(x, ws_ref[...], preferred_element_type=jnp.float32)
           + jnp.dot(mean, wn_ref[...], preferred_element_type=jnp.float32)
           + b_ref[...])
    if relu:
        out = jnp.maximum(out, 0.0)
    o_ref[...] = out


def _mm(relu, x, a0, a1, d0, d1, ws, wn, b):
    R = 1000
    grid = (N // R,)
    return pl.pallas_call(
        functools.partial(_mm_body, relu),
        grid=grid,
        in_specs=[
            pl.BlockSpec((R, D), lambda i: (i, 0)),
            pl.BlockSpec((R, D), lambda i: (i, 0)),
            pl.BlockSpec((R, D), lambda i: (i, 0)),
            pl.BlockSpec((R, 1), lambda i: (i, 0)),
            pl.BlockSpec((R, 1), lambda i: (i, 0)),
            pl.BlockSpec((D, D), lambda i: (0, 0)),
            pl.BlockSpec((D, D), lambda i: (0, 0)),
            pl.BlockSpec((1, D), lambda i: (0, 0)),
        ],
        out_specs=pl.BlockSpec((R, D), lambda i: (i, 0)),
        out_shape=jax.ShapeDtypeStruct((N, D), jnp.float32),
    )(x, a0, a1, d0, d1, ws, wn, b)


def kernel(input_features, edge_index0, edge_index1,
           W_self0, W_neigh0, b0, W_self1, W_neigh1, b1):
    src0 = edge_index0[0].astype(jnp.int32)
    dst0 = edge_index0[1].astype(jnp.int32)
    src1 = edge_index1[0].astype(jnp.int32)
    dst1 = edge_index1[1].astype(jnp.int32)

    sc_agg = _make_sc_agg()
    acc0, deg0 = sc_agg(input_features, src0, dst0)
    h1 = _mm(True, input_features, acc0[0, :N], acc0[1, :N],
             deg0[0, :N].reshape(N, 1), deg0[1, :N].reshape(N, 1),
             W_self0, W_neigh0, b0.reshape(1, D))
    acc1, deg1 = sc_agg(h1, src1, dst1)
    return _mm(False, h1, acc1[0, :N], acc1[1, :N],
               deg1[0, :N].reshape(N, 1), deg1[1, :N].reshape(N, 1),
               W_self1, W_neigh1, b1.reshape(1, D))
